# baseline (speedup 1.0000x reference)
import functools

import numpy as np
import jax
import jax.numpy as jnp
from jax import lax
from jax.experimental import pallas as pl
from jax.experimental.pallas import tpu as pltpu
from jax.experimental.pallas import tpu_sc as plsc

_N, _E, _D, _H = 10000, 320000, 128, 128
_ALPHA = 0.1
_BETA1 = float(np.log(1.5))
_BETA2 = float(np.log(1.25))


def _lstm_gates(g, c):
    i = jax.nn.sigmoid(g[:, 0:_H])
    f = jax.nn.sigmoid(g[:, _H:2 * _H])
    gg = jnp.tanh(g[:, 2 * _H:3 * _H])
    o = jax.nn.sigmoid(g[:, 3 * _H:4 * _H])
    c_new = f * c + i * gg
    return o * jnp.tanh(c_new), c_new


def _w1_chain(w0, wiht, b):
    def step(w, c):
        g = jnp.dot(w, wiht[:, :], preferred_element_type=jnp.float32) + b[:, :]
        return _lstm_gates(g, c)

    h0, c0 = step(w0[:, :], jnp.zeros((_H, _H), jnp.float32))
    h1, _ = step(h0, c0)
    return h0, h1


def _prep_body(x, w, deg, hp, dinv):
    di = lax.rsqrt(deg[0, :_N, :] + 1.0)
    dinv[0, :, :] = di
    h = jnp.dot(x[0, :, :], w[:, :], preferred_element_type=jnp.float32)
    hp[0, :, :] = di * h


def _prep(x01, W, deg):
    return pl.pallas_call(
        _prep_body,
        grid=(2,),
        in_specs=[
            pl.BlockSpec((1, _N, _D), lambda t: (t, 0, 0)),
            pl.BlockSpec((_H, _H), lambda t: (0, 0)),
            pl.BlockSpec((1, _NPAD, 1), lambda t: (t, 0, 0)),
        ],
        out_specs=(
            pl.BlockSpec((1, _N, _D), lambda t: (t, 0, 0)),
            pl.BlockSpec((1, _N, 1), lambda t: (t, 0, 0)),
        ),
        out_shape=(
            jax.ShapeDtypeStruct((2, _N, _D), jnp.float32),
            jax.ShapeDtypeStruct((2, _N, 1), jnp.float32),
        ),
    )(x01, W, deg)


_RB = 2000


def _conv1_body(agg, hp, dinv, b, z1):
    z1[0, :, :] = dinv[0, :, :] * (agg[0, :, :] + hp[0, :, :]) + b[:, :]


def _conv1_finish(agg, hp, dinv, b):
    return pl.pallas_call(
        _conv1_body,
        grid=(2, _N // _RB),
        in_specs=[
            pl.BlockSpec((1, _RB, _D), lambda t, i: (t, i, 0)),
            pl.BlockSpec((1, _RB, _D), lambda t, i: (t, i, 0)),
            pl.BlockSpec((1, _RB, 1), lambda t, i: (t, i, 0)),
            pl.BlockSpec((1, _D), lambda t, i: (0, 0)),
        ],
        out_specs=pl.BlockSpec((1, _RB, _D), lambda t, i: (t, i, 0)),
        out_shape=jax.ShapeDtypeStruct((2, _N, _D), jnp.float32),
    )(agg, hp, dinv, b.reshape(1, _D))


def _conv2a_body(agg, x0, w10, w1iht, b1, gamma, beta, z2):
    t = pl.program_id(0)
    a0, a1 = _w1_chain(w10, w1iht, b1)
    w1a = jnp.where(t == 0, a0, a1)
    out = (1.0 - _ALPHA) * agg[0, :_N, :] + _ALPHA * x0[0, :, :]
    z = (1.0 - _BETA1) * out + _BETA1 * jnp.dot(
        out, w1a, preferred_element_type=jnp.float32)
    mu = jnp.mean(z, axis=0, keepdims=True)
    var = jnp.mean((z - mu) ** 2, axis=0, keepdims=True)
    zn = (z - mu) / jnp.sqrt(var + 1e-5) * gamma[:, :] + beta[:, :]
    z2[0, :, :] = jnp.maximum(zn, 0.0)


def _conv2a(agg, x0, w1_0, r1_Wih, r1_b, gamma, beta):
    return pl.pallas_call(
        _conv2a_body,
        grid=(2,),
        in_specs=[
            pl.BlockSpec((1, _NPAD, _D), lambda t: (t, 0, 0)),
            pl.BlockSpec((1, _N, _D), lambda t: (t, 0, 0)),
            pl.BlockSpec((_H, _H), lambda t: (0, 0)),
            pl.BlockSpec((_H, 4 * _H), lambda t: (0, 0)),
            pl.BlockSpec((1, 4 * _H), lambda t: (0, 0)),
            pl.BlockSpec((1, _D), lambda t: (0, 0)),
            pl.BlockSpec((1, _D), lambda t: (0, 0)),
        ],
        out_specs=pl.BlockSpec((1, _N, _D), lambda t: (t, 0, 0)),
        out_shape=jax.ShapeDtypeStruct((2, _N, _D), jnp.float32),
    )(agg, x0, w1_0, r1_Wih.T, r1_b.reshape(1, 4 * _H),
      gamma.reshape(1, _D), beta.reshape(1, _D))


def _tail_body(agg0, agg1, x00, x01v, w11, w2iht, b2, fwiht, fwhht, fb,
               z3a, z3b, h3):
    b0, b1 = _w1_chain(w11, w2iht, b2)

    def conv2b(agg, x0, w1b):
        out = (1.0 - _ALPHA) * agg[0, :, :] + _ALPHA * x0[0, :, :]
        return (1.0 - _BETA2) * out + _BETA2 * jnp.dot(
            out, w1b, preferred_element_type=jnp.float32)

    za = conv2b(agg0, x00, b0)
    zb = conv2b(agg1, x01v, b1)
    z3a[:, :] = za
    z3b[:, :] = zb

    zero = jnp.zeros_like(za)

    def step(x, h, c):
        g = jnp.dot(x, fwiht[:, :], preferred_element_type=jnp.float32) + fb[:, :]
        g = g + jnp.dot(h, fwhht[:, :], preferred_element_type=jnp.float32)
        return _lstm_gates(g, c)

    h1, c1 = step(za, zero, zero)
    h2, c2 = step(zb, h1, c1)
    hf, _ = step(h2, zero, c2)
    h3[:, :] = hf


def _tail(agg, z1, w1_1, r2_Wih, r2_b, f_Wih, f_Whh, f_b):
    nb = 5
    rb = _N // nb
    return pl.pallas_call(
        _tail_body,
        grid=(nb,),
        in_specs=[
            pl.BlockSpec((1, rb, _D), lambda i: (0, i, 0)),
            pl.BlockSpec((1, rb, _D), lambda i: (1, i, 0)),
            pl.BlockSpec((1, rb, _D), lambda i: (0, i, 0)),
            pl.BlockSpec((1, rb, _D), lambda i: (1, i, 0)),
            pl.BlockSpec((_H, _H), lambda i: (0, 0)),
            pl.BlockSpec((_H, 4 * _H), lambda i: (0, 0)),
            pl.BlockSpec((1, 4 * _H), lambda i: (0, 0)),
            pl.BlockSpec((_H, 4 * _H), lambda i: (0, 0)),
            pl.BlockSpec((_H, 4 * _H), lambda i: (0, 0)),
            pl.BlockSpec((1, 4 * _H), lambda i: (0, 0)),
        ],
        out_specs=(
            pl.BlockSpec((rb, _D), lambda i: (i, 0)),
            pl.BlockSpec((rb, _D), lambda i: (i, 0)),
            pl.BlockSpec((rb, _D), lambda i: (i, 0)),
        ),
        out_shape=(
            jax.ShapeDtypeStruct((_N, _D), jnp.float32),
            jax.ShapeDtypeStruct((_N, _D), jnp.float32),
            jax.ShapeDtypeStruct((_N, _D), jnp.float32),
        ),
    )(agg, agg, z1, z1, w1_1, r2_Wih.T, r2_b.reshape(1, 4 * _H),
      f_Wih.T, f_Whh.T, f_b.reshape(1, 4 * _H))


_NC, _NS = 2, 16
_CHP = 2560
_CPT = _CHP // _NS
_EPAD = _CHP * 128 - _E
_NPAD = 10240
_RPT = _NPAD // _NS

_sc_mesh = plsc.VectorSubcoreMesh(
    core_axis_name="c", subcore_axis_name="s", num_cores=_NC, num_subcores=_NS)


_SLAB = 16
_NSLAB = _CPT // _SLAB


@functools.partial(
    pl.kernel,
    out_type=jax.ShapeDtypeStruct((2 * _NPAD, _D), jnp.float32),
    mesh=_sc_mesh,
    scratch_types=[
        pltpu.VMEM((_SLAB, 128), jnp.int32),
        pltpu.VMEM((_SLAB, 128), jnp.int32),
        pltpu.VMEM((128, _D), jnp.float32),
        pltpu.VMEM((128, _D), jnp.float32),
        pltpu.VMEM_SHARED((_NPAD, _D), jnp.float32),
        pltpu.SemaphoreType.DMA,
        pltpu.SemaphoreType.DMA,
        pltpu.SemaphoreType.DMA,
        pltpu.SemaphoreType.DMA,
    ],
)
def _sc_seg_kernel(table01, src2, dst2, zeros, out, src_v, dst_v,
                   rows_a, rows_b, acc, sem_a, sem_b, sem_sa, sem_sb):
    c = lax.axis_index("c")
    s = lax.axis_index("s")

    pltpu.sync_copy(zeros.at[pl.ds(s * _RPT, _RPT)],
                    acc.at[pl.ds(s * _RPT, _RPT)])

    plsc.subcore_barrier()

    base = c * _CHP + s * _CPT

    def slab(t, _):
        pltpu.sync_copy(src2.at[pl.ds(base + t * _SLAB, _SLAB)], src_v)
        pltpu.sync_copy(dst2.at[pl.ds(base + t * _SLAB, _SLAB)], dst_v)
        pltpu.async_copy(table01.at[src_v.at[0]], rows_a, sem_a)

        def pair(i, _):
            j = 2 * i

            @pl.when(i > 0)
            def _reuse_b():
                pltpu.make_async_copy(
                    rows_b, acc.at[dst_v.at[0]], sem_sb).wait()

            pltpu.async_copy(table01.at[src_v.at[j + 1]], rows_b, sem_b)
            pltpu.make_async_copy(table01.at[src_v.at[j]], rows_a, sem_a).wait()
            pltpu.async_copy(rows_a, acc.at[dst_v.at[j]], sem_sa, add=True)
            pltpu.make_async_copy(table01.at[src_v.at[j + 1]], rows_b,
                                  sem_b).wait()
            pltpu.async_copy(rows_b, acc.at[dst_v.at[j + 1]], sem_sb, add=True)

            @pl.when(i < _SLAB // 2 - 1)
            def _next():
                pltpu.make_async_copy(
                    rows_a, acc.at[dst_v.at[0]], sem_sa).wait()
                pltpu.async_copy(table01.at[src_v.at[j + 2]], rows_a, sem_a)

            return ()

        lax.fori_loop(0, _SLAB // 2, pair, ())
        pltpu.make_async_copy(rows_a, acc.at[dst_v.at[0]], sem_sa).wait()
        pltpu.make_async_copy(rows_b, acc.at[dst_v.at[0]], sem_sb).wait()
        return ()

    lax.fori_loop(0, _NSLAB, slab, ())

    plsc.subcore_barrier()

    pltpu.sync_copy(acc.at[pl.ds(s * _RPT, _RPT)],
                    out.at[pl.ds(c * _NPAD + s * _RPT, _RPT)])


def _seg_rows(table01_flat, src2, dst2, zeros_nd):
    return _sc_seg_kernel(table01_flat, src2, dst2, zeros_nd) \
        .reshape(2, _NPAD, _D)


@functools.partial(
    pl.kernel,
    out_type=jax.ShapeDtypeStruct((2 * _NPAD,), jnp.float32),
    mesh=_sc_mesh,
    scratch_types=[
        pltpu.VMEM((_CPT, 128), jnp.int32),
        pltpu.VMEM((128,), jnp.float32),
        pltpu.VMEM_SHARED((_NPAD,), jnp.float32),
    ],
)
def _sc_deg_kernel(dst2, zeros_n, out, dst_v, ones_v, acc):
    c = lax.axis_index("c")
    s = lax.axis_index("s")

    for k in range(8):
        ones_v[pl.ds(16 * k, 16)] = jnp.full((16,), 1.0, jnp.float32)

    @pl.when(s == 0)
    def _init():
        pltpu.sync_copy(zeros_n, acc)

    pltpu.sync_copy(dst2.at[pl.ds(c * _CHP + s * _CPT, _CPT)], dst_v)

    plsc.subcore_barrier()

    def chunk(j, _):
        pltpu.sync_copy(ones_v, acc.at[dst_v.at[j]], add=True)
        return ()

    lax.fori_loop(0, _CPT, chunk, ())

    plsc.subcore_barrier()

    @pl.when(s == 0)
    def _writeout():
        pltpu.sync_copy(acc, out.at[pl.ds(c * _NPAD, _NPAD)])


def _seg_counts(dst01_2d, zeros_n):
    return _sc_deg_kernel(dst01_2d, zeros_n).reshape(2, _NPAD, 1)


def kernel(x_seq, edge_index_seq, W_gcn, b_gcn, w1_0, w1_1,
           r0_Wih, r0_Whh, r0_bih, r0_bhh,
           r1_Wih, r1_Whh, r1_bih, r1_bhh,
           r2_Wih, r2_Whh, r2_bih, r2_bhh,
           f_Wih, f_Whh, f_bih, f_bhh, bn_gamma, bn_beta):
    zeros_nd = jnp.zeros((_NPAD, _D), jnp.float32)
    zeros_n = jnp.zeros((_NPAD,), jnp.float32)
    pad_src = (jnp.arange(_EPAD, dtype=jnp.int32) * 37) % _N
    pad_dst = _N + (jnp.arange(_EPAD, dtype=jnp.int32) % (_NPAD - _N))

    src01_2d = jnp.concatenate([
        edge_index_seq[0, 0], pad_src,
        edge_index_seq[1, 0] + _N, pad_src + _N,
    ]).reshape(2 * _CHP, 128)
    dst01_2d = jnp.concatenate([
        edge_index_seq[0, 1], pad_dst,
        edge_index_seq[1, 1], pad_dst,
    ]).reshape(2 * _CHP, 128)

    deg = _seg_counts(dst01_2d, zeros_n)

    hp, dinv = _prep(x_seq[:2], W_gcn, deg)
    agg = _seg_rows(hp.reshape(2 * _N, _D), src01_2d, dst01_2d, zeros_nd)
    z1 = _conv1_finish(agg, hp, dinv, b_gcn)
    agg = _seg_rows(z1.reshape(2 * _N, _D), src01_2d, dst01_2d, zeros_nd)
    z2 = _conv2a(agg, z1, w1_0, r1_Wih, r1_bih + r1_bhh, bn_gamma, bn_beta)
    agg = _seg_rows(z2.reshape(2 * _N, _D), src01_2d, dst01_2d, zeros_nd)
    z3a, z3b, h3 = _tail(agg, z1, w1_1, r2_Wih, r2_bih + r2_bhh,
                         f_Wih, f_Whh, f_bih + f_bhh)
    return jnp.stack([z3a, z3b, h3], axis=0)


# speedup vs baseline: 1.2672x; 1.2672x over previous
import functools

import numpy as np
import jax
import jax.numpy as jnp
from jax import lax
from jax.experimental import pallas as pl
from jax.experimental.pallas import tpu as pltpu
from jax.experimental.pallas import tpu_sc as plsc

_N, _E, _D, _H = 10000, 320000, 128, 128
_ALPHA = 0.1
_BETA1 = float(np.log(1.5))
_BETA2 = float(np.log(1.25))


def _lstm_gates(g, c):
    i = jax.nn.sigmoid(g[:, 0:_H])
    f = jax.nn.sigmoid(g[:, _H:2 * _H])
    gg = jnp.tanh(g[:, 2 * _H:3 * _H])
    o = jax.nn.sigmoid(g[:, 3 * _H:4 * _H])
    c_new = f * c + i * gg
    return o * jnp.tanh(c_new), c_new


def _w1_chain(w0, wiht, b):
    def step(w, c):
        g = jnp.dot(w, wiht[:, :], preferred_element_type=jnp.float32) + b[:, :]
        return _lstm_gates(g, c)

    h0, c0 = step(w0[:, :], jnp.zeros((_H, _H), jnp.float32))
    h1, _ = step(h0, c0)
    return h0, h1


def _prep_body(x, w, deg, hp, dinv):
    di = lax.rsqrt(deg[0, :_N, :] + 1.0)
    dinv[0, :, :] = di
    h = jnp.dot(x[0, :, :], w[:, :], preferred_element_type=jnp.float32)
    hp[0, :, :] = di * h


def _prep(x01, W, deg):
    return pl.pallas_call(
        _prep_body,
        grid=(2,),
        in_specs=[
            pl.BlockSpec((1, _N, _D), lambda t: (t, 0, 0)),
            pl.BlockSpec((_H, _H), lambda t: (0, 0)),
            pl.BlockSpec((1, _NPAD, 1), lambda t: (t, 0, 0)),
        ],
        out_specs=(
            pl.BlockSpec((1, _N, _D), lambda t: (t, 0, 0)),
            pl.BlockSpec((1, _N, 1), lambda t: (t, 0, 0)),
        ),
        out_shape=(
            jax.ShapeDtypeStruct((2, _N, _D), jnp.float32),
            jax.ShapeDtypeStruct((2, _N, 1), jnp.float32),
        ),
    )(x01, W, deg)


_RB = 2000


def _conv1_body(agg, hp, dinv, b, z1):
    z1[0, :, :] = dinv[0, :, :] * (agg[0, :, :] + hp[0, :, :]) + b[:, :]


def _conv1_finish(agg, hp, dinv, b):
    return pl.pallas_call(
        _conv1_body,
        grid=(2, _N // _RB),
        in_specs=[
            pl.BlockSpec((1, _RB, _D), lambda t, i: (t, i, 0)),
            pl.BlockSpec((1, _RB, _D), lambda t, i: (t, i, 0)),
            pl.BlockSpec((1, _RB, 1), lambda t, i: (t, i, 0)),
            pl.BlockSpec((1, _D), lambda t, i: (0, 0)),
        ],
        out_specs=pl.BlockSpec((1, _RB, _D), lambda t, i: (t, i, 0)),
        out_shape=jax.ShapeDtypeStruct((2, _N, _D), jnp.float32),
    )(agg, hp, dinv, b.reshape(1, _D))


def _conv2a_body(agg, x0, w10, w1iht, b1, gamma, beta, z2):
    t = pl.program_id(0)
    a0, a1 = _w1_chain(w10, w1iht, b1)
    w1a = jnp.where(t == 0, a0, a1)
    out = (1.0 - _ALPHA) * agg[0, :_N, :] + _ALPHA * x0[0, :, :]
    z = (1.0 - _BETA1) * out + _BETA1 * jnp.dot(
        out, w1a, preferred_element_type=jnp.float32)
    mu = jnp.mean(z, axis=0, keepdims=True)
    var = jnp.mean((z - mu) ** 2, axis=0, keepdims=True)
    zn = (z - mu) / jnp.sqrt(var + 1e-5) * gamma[:, :] + beta[:, :]
    z2[0, :, :] = jnp.maximum(zn, 0.0)


def _conv2a(agg, x0, w1_0, r1_Wih, r1_b, gamma, beta):
    return pl.pallas_call(
        _conv2a_body,
        grid=(2,),
        in_specs=[
            pl.BlockSpec((1, _NPAD, _D), lambda t: (t, 0, 0)),
            pl.BlockSpec((1, _N, _D), lambda t: (t, 0, 0)),
            pl.BlockSpec((_H, _H), lambda t: (0, 0)),
            pl.BlockSpec((_H, 4 * _H), lambda t: (0, 0)),
            pl.BlockSpec((1, 4 * _H), lambda t: (0, 0)),
            pl.BlockSpec((1, _D), lambda t: (0, 0)),
            pl.BlockSpec((1, _D), lambda t: (0, 0)),
        ],
        out_specs=pl.BlockSpec((1, _N, _D), lambda t: (t, 0, 0)),
        out_shape=jax.ShapeDtypeStruct((2, _N, _D), jnp.float32),
    )(agg, x0, w1_0, r1_Wih.T, r1_b.reshape(1, 4 * _H),
      gamma.reshape(1, _D), beta.reshape(1, _D))


def _tail_body(agg0, agg1, x00, x01v, w11, w2iht, b2, fwiht, fwhht, fb, out):
    b0, b1 = _w1_chain(w11, w2iht, b2)

    def conv2b(agg, x0, w1b):
        o = (1.0 - _ALPHA) * agg[0, :, :] + _ALPHA * x0[0, :, :]
        return (1.0 - _BETA2) * o + _BETA2 * jnp.dot(
            o, w1b, preferred_element_type=jnp.float32)

    za = conv2b(agg0, x00, b0)
    zb = conv2b(agg1, x01v, b1)
    out[0, :, :] = za
    out[1, :, :] = zb

    zero = jnp.zeros_like(za)

    def step(x, h, c):
        g = jnp.dot(x, fwiht[:, :], preferred_element_type=jnp.float32) + fb[:, :]
        g = g + jnp.dot(h, fwhht[:, :], preferred_element_type=jnp.float32)
        return _lstm_gates(g, c)

    h1, c1 = step(za, zero, zero)
    h2, c2 = step(zb, h1, c1)
    hf, _ = step(h2, zero, c2)
    out[2, :, :] = hf


def _tail(agg, z1, w1_1, r2_Wih, r2_b, f_Wih, f_Whh, f_b):
    nb = 5
    rb = _N // nb
    return pl.pallas_call(
        _tail_body,
        grid=(nb,),
        in_specs=[
            pl.BlockSpec((1, rb, _D), lambda i: (0, i, 0)),
            pl.BlockSpec((1, rb, _D), lambda i: (1, i, 0)),
            pl.BlockSpec((1, rb, _D), lambda i: (0, i, 0)),
            pl.BlockSpec((1, rb, _D), lambda i: (1, i, 0)),
            pl.BlockSpec((_H, _H), lambda i: (0, 0)),
            pl.BlockSpec((_H, 4 * _H), lambda i: (0, 0)),
            pl.BlockSpec((1, 4 * _H), lambda i: (0, 0)),
            pl.BlockSpec((_H, 4 * _H), lambda i: (0, 0)),
            pl.BlockSpec((_H, 4 * _H), lambda i: (0, 0)),
            pl.BlockSpec((1, 4 * _H), lambda i: (0, 0)),
        ],
        out_specs=pl.BlockSpec((3, rb, _D), lambda i: (0, i, 0)),
        out_shape=jax.ShapeDtypeStruct((3, _N, _D), jnp.float32),
    )(agg, agg, z1, z1, w1_1, r2_Wih.T, r2_b.reshape(1, 4 * _H),
      f_Wih.T, f_Whh.T, f_b.reshape(1, 4 * _H))


_NC, _NS = 2, 16
_CHP = 2560
_CPT = _CHP // _NS
_EPAD = _CHP * 128 - _E
_NPAD = 10240
_RPT = _NPAD // _NS

_sc_mesh = plsc.VectorSubcoreMesh(
    core_axis_name="c", subcore_axis_name="s", num_cores=_NC, num_subcores=_NS)


_SLAB = 16
_NSLAB = _CPT // _SLAB


@functools.partial(
    pl.kernel,
    out_type=jax.ShapeDtypeStruct((2 * _NPAD, _D), jnp.float32),
    mesh=_sc_mesh,
    scratch_types=[
        pltpu.VMEM((_SLAB, 128), jnp.int32),
        pltpu.VMEM((_SLAB, 128), jnp.int32),
        pltpu.VMEM((128, _D), jnp.float32),
        pltpu.VMEM((128, _D), jnp.float32),
        pltpu.VMEM_SHARED((_NPAD, _D), jnp.float32),
        pltpu.SemaphoreType.DMA,
        pltpu.SemaphoreType.DMA,
    ],
)
def _sc_seg_kernel(table01, src2, dst2, zeros, out, src_v, dst_v,
                   rows_a, rows_b, acc, sem_a, sem_b):
    c = lax.axis_index("c")
    s = lax.axis_index("s")

    pltpu.sync_copy(zeros, rows_a)
    for k in range(_RPT // 128):
        pltpu.sync_copy(rows_a, acc.at[pl.ds(s * _RPT + k * 128, 128)])

    plsc.subcore_barrier()

    base = c * _CHP + s * _CPT

    def slab(t, _):
        pltpu.sync_copy(src2.at[pl.ds(base + t * _SLAB, _SLAB)], src_v)
        pltpu.sync_copy(dst2.at[pl.ds(base + t * _SLAB, _SLAB)], dst_v)
        pltpu.async_copy(table01.at[src_v.at[0]], rows_a, sem_a)

        def pair(i, _):
            j = 2 * i
            pltpu.async_copy(table01.at[src_v.at[j + 1]], rows_b, sem_b)
            pltpu.make_async_copy(table01.at[src_v.at[j]], rows_a, sem_a).wait()
            pltpu.sync_copy(rows_a, acc.at[dst_v.at[j]], add=True)

            @pl.when(i < _SLAB // 2 - 1)
            def _next():
                pltpu.async_copy(table01.at[src_v.at[j + 2]], rows_a, sem_a)

            pltpu.make_async_copy(table01.at[src_v.at[j + 1]], rows_b,
                                  sem_b).wait()
            pltpu.sync_copy(rows_b, acc.at[dst_v.at[j + 1]], add=True)
            return ()

        lax.fori_loop(0, _SLAB // 2, pair, ())
        return ()

    lax.fori_loop(0, _NSLAB, slab, ())

    plsc.subcore_barrier()

    pltpu.sync_copy(acc.at[pl.ds(s * _RPT, _RPT)],
                    out.at[pl.ds(c * _NPAD + s * _RPT, _RPT)])


def _seg_rows(table01_flat, src2, dst2, zeros_nd):
    return _sc_seg_kernel(table01_flat, src2, dst2, zeros_nd) \
        .reshape(2, _NPAD, _D)


@functools.partial(
    pl.kernel,
    out_type=jax.ShapeDtypeStruct((2 * _NPAD,), jnp.float32),
    mesh=_sc_mesh,
    scratch_types=[
        pltpu.VMEM((_CPT, 128), jnp.int32),
        pltpu.VMEM((128,), jnp.float32),
        pltpu.VMEM_SHARED((_NPAD,), jnp.float32),
    ],
)
def _sc_deg_kernel(dst2, zeros_n, out, dst_v, ones_v, acc):
    c = lax.axis_index("c")
    s = lax.axis_index("s")

    for k in range(8):
        ones_v[pl.ds(16 * k, 16)] = jnp.full((16,), 1.0, jnp.float32)

    @pl.when(s == 0)
    def _init():
        pltpu.sync_copy(zeros_n, acc)

    pltpu.sync_copy(dst2.at[pl.ds(c * _CHP + s * _CPT, _CPT)], dst_v)

    plsc.subcore_barrier()

    def chunk(j, _):
        pltpu.sync_copy(ones_v, acc.at[dst_v.at[j]], add=True)
        return ()

    lax.fori_loop(0, _CPT, chunk, ())

    plsc.subcore_barrier()

    @pl.when(s == 0)
    def _writeout():
        pltpu.sync_copy(acc, out.at[pl.ds(c * _NPAD, _NPAD)])


def _seg_counts(dst01_2d, zeros_n):
    return _sc_deg_kernel(dst01_2d, zeros_n).reshape(2, _NPAD, 1)


def kernel(x_seq, edge_index_seq, W_gcn, b_gcn, w1_0, w1_1,
           r0_Wih, r0_Whh, r0_bih, r0_bhh,
           r1_Wih, r1_Whh, r1_bih, r1_bhh,
           r2_Wih, r2_Whh, r2_bih, r2_bhh,
           f_Wih, f_Whh, f_bih, f_bhh, bn_gamma, bn_beta):
    zeros_nd = jnp.zeros((128, _D), jnp.float32)
    zeros_n = jnp.zeros((_NPAD,), jnp.float32)
    pad_src = (jnp.arange(_EPAD, dtype=jnp.int32) * 37) % _N
    pad_dst = _N + (jnp.arange(_EPAD, dtype=jnp.int32) % (_NPAD - _N))

    src01_2d = jnp.concatenate([
        edge_index_seq[0, 0], pad_src,
        edge_index_seq[1, 0] + _N, pad_src + _N,
    ]).reshape(2 * _CHP, 128)
    dst01_2d = jnp.concatenate([
        edge_index_seq[0, 1], pad_dst,
        edge_index_seq[1, 1], pad_dst,
    ]).reshape(2 * _CHP, 128)

    deg = _seg_counts(dst01_2d, zeros_n)

    hp, dinv = _prep(x_seq[:2], W_gcn, deg)
    agg = _seg_rows(hp.reshape(2 * _N, _D), src01_2d, dst01_2d, zeros_nd)
    z1 = _conv1_finish(agg, hp, dinv, b_gcn)
    agg = _seg_rows(z1.reshape(2 * _N, _D), src01_2d, dst01_2d, zeros_nd)
    z2 = _conv2a(agg, z1, w1_0, r1_Wih, r1_bih + r1_bhh, bn_gamma, bn_beta)
    agg = _seg_rows(z2.reshape(2 * _N, _D), src01_2d, dst01_2d, zeros_nd)
    return _tail(agg, z1, w1_1, r2_Wih, r2_bih + r2_bhh,
                 f_Wih, f_Whh, f_bih + f_bhh)


# speedup vs baseline: 1.3434x; 1.0601x over previous
import functools

import numpy as np
import jax
import jax.numpy as jnp
from jax import lax
from jax.experimental import pallas as pl
from jax.experimental.pallas import tpu as pltpu
from jax.experimental.pallas import tpu_sc as plsc

_N, _E, _D, _H = 10000, 320000, 128, 128
_ALPHA = 0.1
_BETA1 = float(np.log(1.5))
_BETA2 = float(np.log(1.25))


def _lstm_gates(g, c):
    i = jax.nn.sigmoid(g[:, 0:_H])
    f = jax.nn.sigmoid(g[:, _H:2 * _H])
    gg = jnp.tanh(g[:, 2 * _H:3 * _H])
    o = jax.nn.sigmoid(g[:, 3 * _H:4 * _H])
    c_new = f * c + i * gg
    return o * jnp.tanh(c_new), c_new


def _w1_chain(w0, wiht, b):
    def step(w, c):
        g = jnp.dot(w, wiht[:, :], preferred_element_type=jnp.float32) + b[:, :]
        return _lstm_gates(g, c)

    h0, c0 = step(w0[:, :], jnp.zeros((_H, _H), jnp.float32))
    h1, _ = step(h0, c0)
    return h0, h1


def _prep_body(x, w, deg, hp, dinv):
    di = lax.rsqrt(deg[0, :_N, :] + 1.0)
    dinv[0, :, :] = di
    h = jnp.dot(x[0, :, :], w[:, :], preferred_element_type=jnp.float32)
    hp[0, :, :] = di * h


def _prep(x01, W, deg):
    return pl.pallas_call(
        _prep_body,
        grid=(2,),
        in_specs=[
            pl.BlockSpec((1, _N, _D), lambda t: (t, 0, 0)),
            pl.BlockSpec((_H, _H), lambda t: (0, 0)),
            pl.BlockSpec((1, _NPAD, 1), lambda t: (t, 0, 0)),
        ],
        out_specs=(
            pl.BlockSpec((1, _N, _D), lambda t: (t, 0, 0)),
            pl.BlockSpec((1, _N, 1), lambda t: (t, 0, 0)),
        ),
        out_shape=(
            jax.ShapeDtypeStruct((2, _N, _D), jnp.float32),
            jax.ShapeDtypeStruct((2, _N, 1), jnp.float32),
        ),
    )(x01, W, deg)


_RB = 2000


def _conv1_body(agg, hp, dinv, b, z1):
    z1[0, :, :] = dinv[0, :, :] * (agg[0, :, :] + hp[0, :, :]) + b[:, :]


def _conv1_finish(agg, hp, dinv, b):
    return pl.pallas_call(
        _conv1_body,
        grid=(2, _N // _RB),
        in_specs=[
            pl.BlockSpec((1, _RB, _D), lambda t, i: (t, i, 0)),
            pl.BlockSpec((1, _RB, _D), lambda t, i: (t, i, 0)),
            pl.BlockSpec((1, _RB, 1), lambda t, i: (t, i, 0)),
            pl.BlockSpec((1, _D), lambda t, i: (0, 0)),
        ],
        out_specs=pl.BlockSpec((1, _RB, _D), lambda t, i: (t, i, 0)),
        out_shape=jax.ShapeDtypeStruct((2, _N, _D), jnp.float32),
    )(agg, hp, dinv, b.reshape(1, _D))


def _conv2a_body(agg, x0, w10, w1iht, b1, gamma, beta, z2):
    t = pl.program_id(0)
    a0, a1 = _w1_chain(w10, w1iht, b1)
    w1a = jnp.where(t == 0, a0, a1)
    out = (1.0 - _ALPHA) * agg[0, :_N, :] + _ALPHA * x0[0, :, :]
    z = (1.0 - _BETA1) * out + _BETA1 * jnp.dot(
        out, w1a, preferred_element_type=jnp.float32)
    mu = jnp.mean(z, axis=0, keepdims=True)
    var = jnp.mean((z - mu) ** 2, axis=0, keepdims=True)
    zn = (z - mu) / jnp.sqrt(var + 1e-5) * gamma[:, :] + beta[:, :]
    z2[0, :, :] = jnp.maximum(zn, 0.0)


def _conv2a(agg, x0, w1_0, r1_Wih, r1_b, gamma, beta):
    return pl.pallas_call(
        _conv2a_body,
        grid=(2,),
        in_specs=[
            pl.BlockSpec((1, _NPAD, _D), lambda t: (t, 0, 0)),
            pl.BlockSpec((1, _N, _D), lambda t: (t, 0, 0)),
            pl.BlockSpec((_H, _H), lambda t: (0, 0)),
            pl.BlockSpec((_H, 4 * _H), lambda t: (0, 0)),
            pl.BlockSpec((1, 4 * _H), lambda t: (0, 0)),
            pl.BlockSpec((1, _D), lambda t: (0, 0)),
            pl.BlockSpec((1, _D), lambda t: (0, 0)),
        ],
        out_specs=pl.BlockSpec((1, _N, _D), lambda t: (t, 0, 0)),
        out_shape=jax.ShapeDtypeStruct((2, _N, _D), jnp.float32),
    )(agg, x0, w1_0, r1_Wih.T, r1_b.reshape(1, 4 * _H),
      gamma.reshape(1, _D), beta.reshape(1, _D))


def _tail_body(agg0, agg1, x00, x01v, w11, w2iht, b2, fwiht, fwhht, fb, out):
    b0, b1 = _w1_chain(w11, w2iht, b2)

    def conv2b(agg, x0, w1b):
        o = (1.0 - _ALPHA) * agg[0, :, :] + _ALPHA * x0[0, :, :]
        return (1.0 - _BETA2) * o + _BETA2 * jnp.dot(
            o, w1b, preferred_element_type=jnp.float32)

    za = conv2b(agg0, x00, b0)
    zb = conv2b(agg1, x01v, b1)
    out[0, :, :] = za
    out[1, :, :] = zb

    zero = jnp.zeros_like(za)

    def step(x, h, c):
        g = jnp.dot(x, fwiht[:, :], preferred_element_type=jnp.float32) + fb[:, :]
        g = g + jnp.dot(h, fwhht[:, :], preferred_element_type=jnp.float32)
        return _lstm_gates(g, c)

    h1, c1 = step(za, zero, zero)
    h2, c2 = step(zb, h1, c1)
    hf, _ = step(h2, zero, c2)
    out[2, :, :] = hf


def _tail(agg, z1, w1_1, r2_Wih, r2_b, f_Wih, f_Whh, f_b):
    nb = 5
    rb = _N // nb
    return pl.pallas_call(
        _tail_body,
        grid=(nb,),
        in_specs=[
            pl.BlockSpec((1, rb, _D), lambda i: (0, i, 0)),
            pl.BlockSpec((1, rb, _D), lambda i: (1, i, 0)),
            pl.BlockSpec((1, rb, _D), lambda i: (0, i, 0)),
            pl.BlockSpec((1, rb, _D), lambda i: (1, i, 0)),
            pl.BlockSpec((_H, _H), lambda i: (0, 0)),
            pl.BlockSpec((_H, 4 * _H), lambda i: (0, 0)),
            pl.BlockSpec((1, 4 * _H), lambda i: (0, 0)),
            pl.BlockSpec((_H, 4 * _H), lambda i: (0, 0)),
            pl.BlockSpec((_H, 4 * _H), lambda i: (0, 0)),
            pl.BlockSpec((1, 4 * _H), lambda i: (0, 0)),
        ],
        out_specs=pl.BlockSpec((3, rb, _D), lambda i: (0, i, 0)),
        out_shape=jax.ShapeDtypeStruct((3, _N, _D), jnp.float32),
    )(agg, agg, z1, z1, w1_1, r2_Wih.T, r2_b.reshape(1, 4 * _H),
      f_Wih.T, f_Whh.T, f_b.reshape(1, 4 * _H))


_NC, _NS = 2, 16
_CHP = 2560
_CPT = _CHP // _NS
_EPAD = _CHP * 128 - _E
_NPAD = 10240
_RPT = _NPAD // _NS

_sc_mesh = plsc.VectorSubcoreMesh(
    core_axis_name="c", subcore_axis_name="s", num_cores=_NC, num_subcores=_NS)


_SLAB = 40
_NSLAB = _CPT // _SLAB


@functools.partial(
    pl.kernel,
    out_type=jax.ShapeDtypeStruct((2 * _NPAD, _D), jnp.float32),
    mesh=_sc_mesh,
    scratch_types=[
        pltpu.VMEM((_SLAB, 128), jnp.int32),
        pltpu.VMEM((_SLAB, 128), jnp.int32),
        pltpu.VMEM((128, _D), jnp.float32),
        pltpu.VMEM((128, _D), jnp.float32),
        pltpu.VMEM_SHARED((_NPAD, _D), jnp.float32),
        pltpu.SemaphoreType.DMA,
        pltpu.SemaphoreType.DMA,
    ],
)
def _sc_seg_kernel(table01, src2, dst2, zeros, out, src_v, dst_v,
                   rows_a, rows_b, acc, sem_a, sem_b):
    c = lax.axis_index("c")
    s = lax.axis_index("s")

    pltpu.sync_copy(zeros, rows_a)
    for k in range(_RPT // 128):
        pltpu.sync_copy(rows_a, acc.at[pl.ds(s * _RPT + k * 128, 128)])

    plsc.subcore_barrier()

    base = c * _CHP + s * _CPT

    def slab(t, _):
        pltpu.sync_copy(src2.at[pl.ds(base + t * _SLAB, _SLAB)], src_v)
        pltpu.sync_copy(dst2.at[pl.ds(base + t * _SLAB, _SLAB)], dst_v)
        pltpu.async_copy(table01.at[src_v.at[0]], rows_a, sem_a)

        def pair(i, _):
            j = 2 * i
            pltpu.async_copy(table01.at[src_v.at[j + 1]], rows_b, sem_b)
            pltpu.make_async_copy(table01.at[src_v.at[j]], rows_a, sem_a).wait()
            pltpu.sync_copy(rows_a, acc.at[dst_v.at[j]], add=True)

            @pl.when(i < _SLAB // 2 - 1)
            def _next():
                pltpu.async_copy(table01.at[src_v.at[j + 2]], rows_a, sem_a)

            pltpu.make_async_copy(table01.at[src_v.at[j + 1]], rows_b,
                                  sem_b).wait()
            pltpu.sync_copy(rows_b, acc.at[dst_v.at[j + 1]], add=True)
            return ()

        lax.fori_loop(0, _SLAB // 2, pair, ())
        return ()

    lax.fori_loop(0, _NSLAB, slab, ())

    plsc.subcore_barrier()

    pltpu.sync_copy(acc.at[pl.ds(s * _RPT, _RPT)],
                    out.at[pl.ds(c * _NPAD + s * _RPT, _RPT)])


def _seg_rows(table01_flat, src2, dst2, zeros_nd):
    return _sc_seg_kernel(table01_flat, src2, dst2, zeros_nd) \
        .reshape(2, _NPAD, _D)


@functools.partial(
    pl.kernel,
    out_type=jax.ShapeDtypeStruct((2 * _NPAD,), jnp.float32),
    mesh=_sc_mesh,
    scratch_types=[
        pltpu.VMEM((_CPT, 128), jnp.int32),
        pltpu.VMEM((128,), jnp.float32),
        pltpu.VMEM_SHARED((_NPAD,), jnp.float32),
    ],
)
def _sc_deg_kernel(dst2, zeros_n, out, dst_v, ones_v, acc):
    c = lax.axis_index("c")
    s = lax.axis_index("s")

    for k in range(8):
        ones_v[pl.ds(16 * k, 16)] = jnp.full((16,), 1.0, jnp.float32)

    @pl.when(s == 0)
    def _init():
        pltpu.sync_copy(zeros_n, acc)

    pltpu.sync_copy(dst2.at[pl.ds(c * _CHP + s * _CPT, _CPT)], dst_v)

    plsc.subcore_barrier()

    def chunk(j, _):
        pltpu.sync_copy(ones_v, acc.at[dst_v.at[j]], add=True)
        return ()

    lax.fori_loop(0, _CPT, chunk, ())

    plsc.subcore_barrier()

    @pl.when(s == 0)
    def _writeout():
        pltpu.sync_copy(acc, out.at[pl.ds(c * _NPAD, _NPAD)])


def _seg_counts(dst01_2d, zeros_n):
    return _sc_deg_kernel(dst01_2d, zeros_n).reshape(2, _NPAD, 1)


def kernel(x_seq, edge_index_seq, W_gcn, b_gcn, w1_0, w1_1,
           r0_Wih, r0_Whh, r0_bih, r0_bhh,
           r1_Wih, r1_Whh, r1_bih, r1_bhh,
           r2_Wih, r2_Whh, r2_bih, r2_bhh,
           f_Wih, f_Whh, f_bih, f_bhh, bn_gamma, bn_beta):
    zeros_nd = jnp.zeros((128, _D), jnp.float32)
    zeros_n = jnp.zeros((_NPAD,), jnp.float32)
    pad_src = (jnp.arange(_EPAD, dtype=jnp.int32) * 37) % _N
    pad_dst = _N + (jnp.arange(_EPAD, dtype=jnp.int32) % (_NPAD - _N))

    src01_2d = jnp.concatenate([
        edge_index_seq[0, 0], pad_src,
        edge_index_seq[1, 0] + _N, pad_src + _N,
    ]).reshape(2 * _CHP, 128)
    dst01_2d = jnp.concatenate([
        edge_index_seq[0, 1], pad_dst,
        edge_index_seq[1, 1], pad_dst,
    ]).reshape(2 * _CHP, 128)

    deg = _seg_counts(dst01_2d, zeros_n)

    hp, dinv = _prep(x_seq[:2], W_gcn, deg)
    agg = _seg_rows(hp.reshape(2 * _N, _D), src01_2d, dst01_2d, zeros_nd)
    z1 = _conv1_finish(agg, hp, dinv, b_gcn)
    agg = _seg_rows(z1.reshape(2 * _N, _D), src01_2d, dst01_2d, zeros_nd)
    z2 = _conv2a(agg, z1, w1_0, r1_Wih, r1_bih + r1_bhh, bn_gamma, bn_beta)
    agg = _seg_rows(z2.reshape(2 * _N, _D), src01_2d, dst01_2d, zeros_nd)
    return _tail(agg, z1, w1_1, r2_Wih, r2_bih + r2_bhh,
                 f_Wih, f_Whh, f_bih + f_bhh)


# speedup vs baseline: 1.3597x; 1.0122x over previous
import functools

import numpy as np
import jax
import jax.numpy as jnp
from jax import lax
from jax.experimental import pallas as pl
from jax.experimental.pallas import tpu as pltpu
from jax.experimental.pallas import tpu_sc as plsc

_N, _E, _D, _H = 10000, 320000, 128, 128
_ALPHA = 0.1
_BETA1 = float(np.log(1.5))
_BETA2 = float(np.log(1.25))


def _lstm_gates(g, c):
    i = jax.nn.sigmoid(g[:, 0:_H])
    f = jax.nn.sigmoid(g[:, _H:2 * _H])
    gg = jnp.tanh(g[:, 2 * _H:3 * _H])
    o = jax.nn.sigmoid(g[:, 3 * _H:4 * _H])
    c_new = f * c + i * gg
    return o * jnp.tanh(c_new), c_new


def _w1_chain(w0, wiht, b):
    def step(w, c):
        g = jnp.dot(w, wiht[:, :], preferred_element_type=jnp.float32) + b[:, :]
        return _lstm_gates(g, c)

    h0, c0 = step(w0[:, :], jnp.zeros((_H, _H), jnp.float32))
    h1, _ = step(h0, c0)
    return h0, h1


def _prep_body(x, w, deg, hp, dinv):
    di = lax.rsqrt(deg[0, :_N, :] + 1.0)
    dinv[0, :, :] = di
    h = jnp.dot(x[0, :, :], w[:, :], preferred_element_type=jnp.float32)
    hp[0, :, :] = di * h


def _prep(x01, W, deg):
    return pl.pallas_call(
        _prep_body,
        grid=(2,),
        in_specs=[
            pl.BlockSpec((1, _N, _D), lambda t: (t, 0, 0)),
            pl.BlockSpec((_H, _H), lambda t: (0, 0)),
            pl.BlockSpec((1, _NPAD, 1), lambda t: (t, 0, 0)),
        ],
        out_specs=(
            pl.BlockSpec((1, _N, _D), lambda t: (t, 0, 0)),
            pl.BlockSpec((1, _N, 1), lambda t: (t, 0, 0)),
        ),
        out_shape=(
            jax.ShapeDtypeStruct((2, _N, _D), jnp.float32),
            jax.ShapeDtypeStruct((2, _N, 1), jnp.float32),
        ),
    )(x01, W, deg)


_RB = 2000


def _conv1_body(agg, hp, dinv, b, z1):
    z1[0, :, :] = dinv[0, :, :] * (agg[0, :, :] + hp[0, :, :]) + b[:, :]


def _conv1_finish(agg, hp, dinv, b):
    return pl.pallas_call(
        _conv1_body,
        grid=(2, _N // _RB),
        in_specs=[
            pl.BlockSpec((1, _RB, _D), lambda t, i: (t, i, 0)),
            pl.BlockSpec((1, _RB, _D), lambda t, i: (t, i, 0)),
            pl.BlockSpec((1, _RB, 1), lambda t, i: (t, i, 0)),
            pl.BlockSpec((1, _D), lambda t, i: (0, 0)),
        ],
        out_specs=pl.BlockSpec((1, _RB, _D), lambda t, i: (t, i, 0)),
        out_shape=jax.ShapeDtypeStruct((2, _N, _D), jnp.float32),
    )(agg, hp, dinv, b.reshape(1, _D))


def _conv2a_body(agg, x0, w10, w1iht, b1, gamma, beta, z2):
    t = pl.program_id(0)
    a0, a1 = _w1_chain(w10, w1iht, b1)
    w1a = jnp.where(t == 0, a0, a1)
    out = (1.0 - _ALPHA) * agg[0, :_N, :] + _ALPHA * x0[0, :, :]
    z = (1.0 - _BETA1) * out + _BETA1 * jnp.dot(
        out, w1a, preferred_element_type=jnp.float32)
    mu = jnp.mean(z, axis=0, keepdims=True)
    var = jnp.mean((z - mu) ** 2, axis=0, keepdims=True)
    zn = (z - mu) / jnp.sqrt(var + 1e-5) * gamma[:, :] + beta[:, :]
    z2[0, :, :] = jnp.maximum(zn, 0.0)


def _conv2a(agg, x0, w1_0, r1_Wih, r1_b, gamma, beta):
    return pl.pallas_call(
        _conv2a_body,
        grid=(2,),
        in_specs=[
            pl.BlockSpec((1, _NPAD, _D), lambda t: (t, 0, 0)),
            pl.BlockSpec((1, _N, _D), lambda t: (t, 0, 0)),
            pl.BlockSpec((_H, _H), lambda t: (0, 0)),
            pl.BlockSpec((_H, 4 * _H), lambda t: (0, 0)),
            pl.BlockSpec((1, 4 * _H), lambda t: (0, 0)),
            pl.BlockSpec((1, _D), lambda t: (0, 0)),
            pl.BlockSpec((1, _D), lambda t: (0, 0)),
        ],
        out_specs=pl.BlockSpec((1, _N, _D), lambda t: (t, 0, 0)),
        out_shape=jax.ShapeDtypeStruct((2, _N, _D), jnp.float32),
    )(agg, x0, w1_0, r1_Wih.T, r1_b.reshape(1, 4 * _H),
      gamma.reshape(1, _D), beta.reshape(1, _D))


def _tail_body(agg0, agg1, x00, x01v, w11, w2iht, b2, fwiht, fwhht, fb, out):
    b0, b1 = _w1_chain(w11, w2iht, b2)

    def conv2b(agg, x0, w1b):
        o = (1.0 - _ALPHA) * agg[0, :, :] + _ALPHA * x0[0, :, :]
        return (1.0 - _BETA2) * o + _BETA2 * jnp.dot(
            o, w1b, preferred_element_type=jnp.float32)

    za = conv2b(agg0, x00, b0)
    zb = conv2b(agg1, x01v, b1)
    out[0, :, :] = za
    out[1, :, :] = zb

    zero = jnp.zeros_like(za)

    def step(x, h, c):
        g = jnp.dot(x, fwiht[:, :], preferred_element_type=jnp.float32) + fb[:, :]
        g = g + jnp.dot(h, fwhht[:, :], preferred_element_type=jnp.float32)
        return _lstm_gates(g, c)

    h1, c1 = step(za, zero, zero)
    h2, c2 = step(zb, h1, c1)
    hf, _ = step(h2, zero, c2)
    out[2, :, :] = hf


def _tail(agg, z1, w1_1, r2_Wih, r2_b, f_Wih, f_Whh, f_b):
    nb = 5
    rb = _N // nb
    return pl.pallas_call(
        _tail_body,
        grid=(nb,),
        in_specs=[
            pl.BlockSpec((1, rb, _D), lambda i: (0, i, 0)),
            pl.BlockSpec((1, rb, _D), lambda i: (1, i, 0)),
            pl.BlockSpec((1, rb, _D), lambda i: (0, i, 0)),
            pl.BlockSpec((1, rb, _D), lambda i: (1, i, 0)),
            pl.BlockSpec((_H, _H), lambda i: (0, 0)),
            pl.BlockSpec((_H, 4 * _H), lambda i: (0, 0)),
            pl.BlockSpec((1, 4 * _H), lambda i: (0, 0)),
            pl.BlockSpec((_H, 4 * _H), lambda i: (0, 0)),
            pl.BlockSpec((_H, 4 * _H), lambda i: (0, 0)),
            pl.BlockSpec((1, 4 * _H), lambda i: (0, 0)),
        ],
        out_specs=pl.BlockSpec((3, rb, _D), lambda i: (0, i, 0)),
        out_shape=jax.ShapeDtypeStruct((3, _N, _D), jnp.float32),
    )(agg, agg, z1, z1, w1_1, r2_Wih.T, r2_b.reshape(1, 4 * _H),
      f_Wih.T, f_Whh.T, f_b.reshape(1, 4 * _H))


_NC, _NS = 2, 16
_CHP = 2560
_CPT = _CHP // _NS
_EPAD = _CHP * 128 - _E
_NPAD = 10240
_RPT = _NPAD // _NS

_sc_mesh = plsc.VectorSubcoreMesh(
    core_axis_name="c", subcore_axis_name="s", num_cores=_NC, num_subcores=_NS)


_SLAB = 40
_NSLAB = _CPT // _SLAB


@functools.partial(
    pl.kernel,
    out_type=jax.ShapeDtypeStruct((2 * _NPAD, _D), jnp.float32),
    mesh=_sc_mesh,
    scratch_types=[
        pltpu.VMEM((_SLAB, 128), jnp.int32),
        pltpu.VMEM((_SLAB, 128), jnp.int32),
        pltpu.VMEM((128, _D), jnp.float32),
        pltpu.VMEM((128, _D), jnp.float32),
        pltpu.VMEM_SHARED((_NPAD, _D), jnp.float32),
        pltpu.SemaphoreType.DMA,
        pltpu.SemaphoreType.DMA,
    ],
)
def _sc_seg_kernel(table01, src2, dst2, zeros, out, src_v, dst_v,
                   rows_a, rows_b, acc, sem_a, sem_b):
    c = lax.axis_index("c")
    s = lax.axis_index("s")

    pltpu.sync_copy(zeros, rows_a)
    for k in range(_RPT // 128):
        pltpu.sync_copy(rows_a, acc.at[pl.ds(s * _RPT + k * 128, 128)])

    plsc.subcore_barrier()

    base = c * _CHP + s * _CPT

    def slab(t, _):
        pltpu.sync_copy(src2.at[pl.ds(base + t * _SLAB, _SLAB)], src_v)
        pltpu.sync_copy(dst2.at[pl.ds(base + t * _SLAB, _SLAB)], dst_v)
        pltpu.async_copy(table01.at[src_v.at[0]], rows_a, sem_a)

        def pair(i, _):
            j = 2 * i
            pltpu.async_copy(table01.at[src_v.at[j + 1]], rows_b, sem_b)
            pltpu.make_async_copy(table01.at[src_v.at[j]], rows_a, sem_a).wait()
            pltpu.sync_copy(rows_a, acc.at[dst_v.at[j]], add=True)

            @pl.when(i < _SLAB // 2 - 1)
            def _next():
                pltpu.async_copy(table01.at[src_v.at[j + 2]], rows_a, sem_a)

            pltpu.make_async_copy(table01.at[src_v.at[j + 1]], rows_b,
                                  sem_b).wait()
            pltpu.sync_copy(rows_b, acc.at[dst_v.at[j + 1]], add=True)
            return ()

        lax.fori_loop(0, _SLAB // 2, pair, ())
        return ()

    lax.fori_loop(0, _NSLAB, slab, ())

    plsc.subcore_barrier()

    pltpu.sync_copy(acc.at[pl.ds(s * _RPT, _RPT)],
                    out.at[pl.ds(c * _NPAD + s * _RPT, _RPT)])


def _seg_rows(table01_flat, src2, dst2, zeros_nd):
    return _sc_seg_kernel(table01_flat, src2, dst2, zeros_nd) \
        .reshape(2, _NPAD, _D)


@functools.partial(
    pl.kernel,
    out_type=jax.ShapeDtypeStruct((2 * _NPAD,), jnp.float32),
    mesh=_sc_mesh,
    scratch_types=[
        pltpu.VMEM((_CPT, 128), jnp.int32),
        pltpu.VMEM((128,), jnp.float32),
        pltpu.VMEM_SHARED((_NPAD,), jnp.float32),
        pltpu.SemaphoreType.DMA,
    ],
)
def _sc_deg_kernel(dst2, zeros_n, out, dst_v, ones_v, acc, sem):
    c = lax.axis_index("c")
    s = lax.axis_index("s")

    for k in range(8):
        ones_v[pl.ds(16 * k, 16)] = jnp.full((16,), 1.0, jnp.float32)

    @pl.when(s == 0)
    def _init():
        pltpu.sync_copy(zeros_n, acc)

    pltpu.sync_copy(dst2.at[pl.ds(c * _CHP + s * _CPT, _CPT)], dst_v)

    plsc.subcore_barrier()

    def batch(i, _):
        for b in range(8):
            pltpu.async_copy(ones_v, acc.at[dst_v.at[8 * i + b]], sem,
                             add=True)
        for b in range(8):
            pltpu.make_async_copy(ones_v, acc.at[dst_v.at[0]], sem).wait()
        return ()

    lax.fori_loop(0, _CPT // 8, batch, ())

    plsc.subcore_barrier()

    @pl.when(s == 0)
    def _writeout():
        pltpu.sync_copy(acc, out.at[pl.ds(c * _NPAD, _NPAD)])


def _seg_counts(dst01_2d, zeros_n):
    return _sc_deg_kernel(dst01_2d, zeros_n).reshape(2, _NPAD, 1)


def kernel(x_seq, edge_index_seq, W_gcn, b_gcn, w1_0, w1_1,
           r0_Wih, r0_Whh, r0_bih, r0_bhh,
           r1_Wih, r1_Whh, r1_bih, r1_bhh,
           r2_Wih, r2_Whh, r2_bih, r2_bhh,
           f_Wih, f_Whh, f_bih, f_bhh, bn_gamma, bn_beta):
    zeros_nd = jnp.zeros((128, _D), jnp.float32)
    zeros_n = jnp.zeros((_NPAD,), jnp.float32)
    pad_src = (jnp.arange(_EPAD, dtype=jnp.int32) * 37) % _N
    pad_dst = _N + (jnp.arange(_EPAD, dtype=jnp.int32) % (_NPAD - _N))

    src01_2d = jnp.concatenate([
        edge_index_seq[0, 0], pad_src,
        edge_index_seq[1, 0] + _N, pad_src + _N,
    ]).reshape(2 * _CHP, 128)
    dst01_2d = jnp.concatenate([
        edge_index_seq[0, 1], pad_dst,
        edge_index_seq[1, 1], pad_dst,
    ]).reshape(2 * _CHP, 128)

    deg = _seg_counts(dst01_2d, zeros_n)

    hp, dinv = _prep(x_seq[:2], W_gcn, deg)
    agg = _seg_rows(hp.reshape(2 * _N, _D), src01_2d, dst01_2d, zeros_nd)
    z1 = _conv1_finish(agg, hp, dinv, b_gcn)
    agg = _seg_rows(z1.reshape(2 * _N, _D), src01_2d, dst01_2d, zeros_nd)
    z2 = _conv2a(agg, z1, w1_0, r1_Wih, r1_bih + r1_bhh, bn_gamma, bn_beta)
    agg = _seg_rows(z2.reshape(2 * _N, _D), src01_2d, dst01_2d, zeros_nd)
    return _tail(agg, z1, w1_1, r2_Wih, r2_bih + r2_bhh,
                 f_Wih, f_Whh, f_bih + f_bhh)
